# baseline (speedup 1.0000x reference)
import functools

import jax
import jax.numpy as jnp
from jax import lax
from jax.experimental import pallas as pl
from jax.experimental.pallas import tpu as pltpu
from jax.experimental.pallas import tpu_sc as plsc

CHUNK = 128
NBUF = 8


@functools.lru_cache(maxsize=None)
def _build(V, D, B_total):
    info = plsc.get_sparse_core_info()
    NC, NS = info.num_cores, info.num_subcores
    NW = NC * NS
    assert B_total % (NW * CHUNK) == 0, (B_total, NW, CHUNK)
    b_per_w = B_total // NW
    n_chunks = b_per_w // CHUNK
    assert n_chunks % NBUF == 0, (n_chunks, NBUF)
    n_groups = n_chunks // NBUF

    mesh = plsc.VectorSubcoreMesh(core_axis_name="c", subcore_axis_name="s")

    @functools.partial(
        pl.kernel,
        mesh=mesh,
        out_type=jax.ShapeDtypeStruct((B_total, D), jnp.float32),
        compiler_params=pltpu.CompilerParams(use_tc_tiling_on_sc=False),
        scratch_types=(
            [pltpu.VMEM((n_chunks, CHUNK), jnp.int32)]
            + [pltpu.VMEM((CHUNK, D), jnp.float32) for _ in range(NBUF)]
            + [pltpu.SemaphoreType.DMA for _ in range(2 * NBUF)]
        ),
    )
    def gather_kernel(table_hbm, idx_hbm, out_hbm, idx_v, *scratch):
        rows = scratch[:NBUF]
        gsem = scratch[NBUF:2 * NBUF]
        wsem = scratch[2 * NBUF:]
        wid = lax.axis_index("s") * NC + lax.axis_index("c")
        base = wid * b_per_w

        pltpu.sync_copy(idx_hbm.at[wid], idx_v)

        def start_gather(j, b):
            pltpu.make_async_copy(
                table_hbm.at[idx_v.at[j]], rows[b], gsem[b]).start()

        def wait_gather(b):
            pltpu.make_async_copy(
                table_hbm.at[idx_v.at[0]], rows[b], gsem[b]).wait()

        def start_write(j, b):
            pltpu.make_async_copy(
                rows[b], out_hbm.at[pl.ds(base + j * CHUNK, CHUNK)],
                wsem[b]).start()

        def wait_write(b):
            pltpu.make_async_copy(
                rows[b], out_hbm.at[pl.ds(base, CHUNK)], wsem[b]).wait()

        for b in range(NBUF):
            start_gather(b, b)

        def group(g, carry):
            j0 = g * NBUF
            for b in range(NBUF):
                wait_gather(b)
                start_write(j0 + b, b)
            for b in range(NBUF):
                wait_write(b)
                start_gather(j0 + NBUF + b, b)
            return carry

        lax.fori_loop(0, n_groups - 1, group, 0)

        j0 = (n_groups - 1) * NBUF
        for b in range(NBUF):
            wait_gather(b)
            start_write(j0 + b, b)
        for b in range(NBUF):
            wait_write(b)

    return gather_kernel, NW, n_chunks


def kernel(x, table):
    V, D = table.shape
    out_shape = x.shape + (D,)
    B_total = 1
    for s in x.shape:
        B_total *= s
    fn, NW, n_chunks = _build(V, D, B_total)
    idx = x.reshape(NW, n_chunks, CHUNK).astype(jnp.int32)
    out = fn(table, idx)
    return out.reshape(out_shape)
